# baseline (speedup 1.0000x reference)
import functools

import jax
import jax.numpy as jnp
from jax import lax
from jax.experimental import pallas as pl
from jax.experimental.pallas import tpu as pltpu
from jax.experimental.pallas import tpu_sc as plsc

_VOCAB = 100000
_D = 64
_S = 8
_B = 4096
_SEQ = 52
_SEQP = 56
_L = 16
_NC = 2
_NS = 16
_NW = _NC * _NS
_BPW = _B // _NW
_KD = _D // _L
_NBUF = 4


@functools.partial(
    pl.kernel,
    mesh=plsc.VectorSubcoreMesh(core_axis_name="c", subcore_axis_name="s"),
    compiler_params=pltpu.CompilerParams(use_tc_tiling_on_sc=True,
                                         needs_layout_passes=False),
    out_type=jax.ShapeDtypeStruct((_NW, _S, 8 * _L), jnp.float32),
    scratch_types=[
        pltpu.VMEM((_BPW, 8 * _L), jnp.int32),
        pltpu.VMEM((_BPW,), jnp.int32),
        pltpu.VMEM((_NBUF, _SEQP, 8 * _L), jnp.float32),
        pltpu.VMEM((_BPW, _S * _D), jnp.float32),
        pltpu.VMEM((_S, 8 * _L), jnp.float32),
        pltpu.SemaphoreType.DMA,
        pltpu.SemaphoreType.DMA,
        pltpu.SemaphoreType.DMA,
        pltpu.SemaphoreType.DMA,
        pltpu.SemaphoreType.DMA,
    ],
)
def _sense_partials(xp_hbm, wgp_hbm, ws2_hbm, out_hbm,
                    xr_v, x0_v, rows_v, ws_v, part_v,
                    sem_ws, sem0, sem1, sem2, sem3):
    wid = lax.axis_index("s") * _NC + lax.axis_index("c")
    sems = (sem0, sem1, sem2, sem3)
    iota = lax.iota(jnp.int32, _L)
    zeros = jnp.zeros((_L,), jnp.float32)

    pltpu.sync_copy(xp_hbm.at[pl.ds(wid * _BPW, _BPW)], xr_v)

    zero16 = jnp.zeros((_L,), jnp.int32)
    for j in range(_BPW // _L):
        rows = jnp.full((_L,), j * _L, jnp.int32) + iota
        x0_v[pl.ds(j * _L, _L)] = plsc.load_gather(xr_v, [rows, zero16])
    pltpu.async_copy(ws2_hbm.at[x0_v], ws_v, sem_ws)

    def start_row(i, buf):
        pltpu.async_copy(wgp_hbm.at[xr_v.at[i, pl.ds(0, _SEQP)]],
                         rows_v.at[buf], sems[buf])

    def wait_row(i, buf):
        pltpu.make_async_copy(wgp_hbm.at[xr_v.at[i, pl.ds(0, _SEQP)]],
                              rows_v.at[buf], sems[buf]).wait()

    for t in range(_NBUF):
        start_row(t, t)

    pltpu.make_async_copy(ws2_hbm.at[x0_v], ws_v, sem_ws).wait()

    def row_chunk(j, accs):
        for t in range(_NBUF):
            i = j * _NBUF + t
            wait_row(i, t)
            a = [rows_v[t, 2, pl.ds(k * _L, _L)] for k in range(_KD)]
            for c in range(3, _SEQ):
                for k in range(_KD):
                    a[k] = a[k] + rows_v[t, c, pl.ds(k * _L, _L)]

            @pl.when(i + _NBUF < _BPW)
            def _():
                start_row(i + _NBUF, t)

            new = []
            for s in range(_S):
                v = accs[s]
                for k in range(_KD):
                    v = v + ws_v[i, pl.ds(s * _D + k * _L, _L)] * a[k]
                new.append(v)
            accs = tuple(new)
        return accs

    accs = lax.fori_loop(0, _BPW // _NBUF, row_chunk,
                         tuple(zeros for _ in range(_S)))

    for s in range(_S):
        for k in range(8):
            part_v[s, pl.ds(k * _L, _L)] = accs[s] if k == 0 else zeros
    pltpu.sync_copy(part_v, out_hbm.at[wid])


@jax.jit
def kernel(x, W_g, W_s):
    xp = jnp.pad(x, ((0, 0), (0, 8 * _L - _SEQ)))
    wgp = jnp.pad(W_g, ((0, 0), (0, _D)))
    ws2 = W_s.reshape(_VOCAB, _S * _D)
    partials = _sense_partials(xp, wgp, ws2)
    return jax.nn.sigmoid(jnp.sum(partials, axis=(0, 2)))


# speedup vs baseline: 3.2281x; 3.2281x over previous
import functools

import jax
import jax.numpy as jnp
from jax import lax
from jax.experimental import pallas as pl
from jax.experimental.pallas import tpu as pltpu
from jax.experimental.pallas import tpu_sc as plsc

_VOCAB = 100000
_D = 64
_S = 8
_B = 4096
_SEQ = 52
_SEQP = 56
_L = 16
_NC = 2
_NS = 16
_NW = _NC * _NS
_BPW = _B // _NW
_KD = _D // _L
_NBUF = 3
_WSC = 32
_NQ = _BPW // _WSC


@functools.partial(
    pl.kernel,
    mesh=plsc.VectorSubcoreMesh(core_axis_name="c", subcore_axis_name="s"),
    compiler_params=pltpu.CompilerParams(use_tc_tiling_on_sc=True,
                                         needs_layout_passes=False),
    out_type=jax.ShapeDtypeStruct((_NW, _S, 8 * _L), jnp.float32),
    scratch_types=[
        pltpu.VMEM((_SEQP, _BPW), jnp.int32),
        pltpu.VMEM((_NBUF, _BPW, 8 * _L), jnp.float32),
        pltpu.VMEM((_BPW, _D), jnp.float32),
        pltpu.VMEM((2, _WSC, _S * _D), jnp.float32),
        pltpu.VMEM((_S, 8 * _L), jnp.float32),
        pltpu.SemaphoreType.DMA,
        pltpu.SemaphoreType.DMA,
        pltpu.SemaphoreType.DMA,
        pltpu.SemaphoreType.DMA,
    ],
)
def _sense_partials(xT_hbm, wgp_hbm, ws2_hbm, out_hbm,
                    x_v, rows_v, acc_v, ws_v, part_v,
                    sem_ws, sem0, sem1, sem2):
    wid = lax.axis_index("s") * _NC + lax.axis_index("c")
    sems = (sem0, sem1, sem2)
    zeros = jnp.zeros((_L,), jnp.float32)

    pltpu.sync_copy(xT_hbm.at[wid], x_v)

    for q in range(2):
        pltpu.async_copy(ws2_hbm.at[x_v.at[0, pl.ds(q * _WSC, _WSC)]],
                         ws_v.at[q], sem_ws)

    def start_col(c, buf):
        pltpu.async_copy(wgp_hbm.at[x_v.at[c]], rows_v.at[buf], sems[buf])

    def wait_col(c, buf):
        pltpu.make_async_copy(
            wgp_hbm.at[x_v.at[c]], rows_v.at[buf], sems[buf]).wait()

    def acc_col(buf, first):
        def body(i, carry):
            for k in range(_KD):
                sl = pl.ds(k * _L, _L)
                v = rows_v[buf, i, sl]
                if first:
                    acc_v[i, sl] = v
                else:
                    plsc.addupdate(acc_v.at[i, sl], v)
            return carry
        lax.fori_loop(0, _BPW, body, 0, unroll=4)

    for t in range(_NBUF):
        start_col(2 + t, t)

    wait_col(2, 0)
    acc_col(0, first=True)
    start_col(5, 0)

    def ring_body(j, carry):
        c0 = 3 + 3 * j
        for t in range(3):
            buf = (1 + t) % _NBUF
            c = c0 + t
            wait_col(c, buf)
            acc_col(buf, first=False)

            @pl.when(c + _NBUF < _SEQ)
            def _():
                start_col(c + _NBUF, buf)
        return carry

    lax.fori_loop(0, 16, ring_body, 0)

    wait_col(51, 1)
    acc_col(1, first=False)

    accs = tuple(zeros for _ in range(_S))
    for q in range(_NQ):
        qb = q % 2
        idxref = x_v.at[0, pl.ds(q * _WSC, _WSC)]
        pltpu.make_async_copy(ws2_hbm.at[idxref], ws_v.at[qb], sem_ws).wait()

        def score_body(i, acc_c, q=q, qb=qb):
            ctx = [acc_v[q * _WSC + i, pl.ds(k * _L, _L)]
                   for k in range(_KD)]
            out = []
            for s in range(_S):
                a = acc_c[s]
                for k in range(_KD):
                    a = a + ws_v[qb, i, pl.ds(s * _D + k * _L, _L)] * ctx[k]
                out.append(a)
            return tuple(out)

        accs = lax.fori_loop(0, _WSC, score_body, accs)

        if q + 2 < _NQ:
            nidx = x_v.at[0, pl.ds((q + 2) * _WSC, _WSC)]
            pltpu.async_copy(ws2_hbm.at[nidx], ws_v.at[qb], sem_ws)

    for s in range(_S):
        for k in range(8):
            part_v[s, pl.ds(k * _L, _L)] = accs[s] if k == 0 else zeros
    pltpu.sync_copy(part_v, out_hbm.at[wid])


@jax.jit
def kernel(x, W_g, W_s):
    xT = jnp.pad(x.T, ((0, _SEQP - _SEQ), (0, 0)))
    xT = xT.reshape(_SEQP, _NW, _BPW).transpose(1, 0, 2)
    wgp = jnp.pad(W_g, ((0, 0), (0, _D)))
    ws2 = W_s.reshape(_VOCAB, _S * _D)
    partials = _sense_partials(xT, wgp, ws2)
    return jax.nn.sigmoid(jnp.sum(partials, axis=(0, 2)))
